# baseline (speedup 1.0000x reference)
import functools

import jax
import jax.numpy as jnp
from jax import lax
from jax.experimental import pallas as pl
from jax.experimental.pallas import tpu as pltpu
from jax.experimental.pallas import tpu_sc as plsc

N_TOKENS = 32768
D_MODEL = 768
N_EXP = 8
BLK = 2048

NC, NS, L = 2, 16, 16
NW = NC * NS
CHUNK = N_TOKENS // NW
GRPS = CHUNK // L


def _matmul_blk(x_ref, w_ref, lt_ref):
    lt_ref[...] = jax.lax.dot_general(
        w_ref[...], x_ref[...], (((1,), (1,)), ((), ())),
        preferred_element_type=jnp.float32)


_sc_mesh = plsc.VectorSubcoreMesh(
    core_axis_name="c", subcore_axis_name="s", num_cores=NC, num_subcores=NS)


@functools.partial(
    pl.kernel,
    mesh=_sc_mesh,
    out_type=[
        jax.ShapeDtypeStruct((N_EXP, N_TOKENS), jnp.float32),
        jax.ShapeDtypeStruct((N_TOKENS,), jnp.float32),
        jax.ShapeDtypeStruct((N_TOKENS,), jnp.float32),
    ],
    scratch_types=[
        pltpu.VMEM((N_EXP, CHUNK), jnp.float32),
        pltpu.VMEM((N_EXP, CHUNK), jnp.float32),
        pltpu.VMEM((CHUNK,), jnp.float32),
        pltpu.VMEM((CHUNK,), jnp.float32),
    ],
)
def _sc_softmax(lt_hbm, probs_hbm, a_hbm, s_hbm,
                lbuf, pbuf, abuf, sbuf):
    wid = lax.axis_index("s") * NC + lax.axis_index("c")
    base = wid * CHUNK
    for e in range(N_EXP):
        pltpu.sync_copy(lt_hbm.at[e, pl.ds(base, CHUNK)], lbuf.at[e])

    def body(g, carry):
        off = g * L
        ls = [lbuf[e, pl.ds(off, L)] for e in range(N_EXP)]
        m = ls[0]
        for e in range(1, N_EXP):
            m = jnp.maximum(m, ls[e])
        es = [jnp.exp(ls[e] - m) for e in range(N_EXP)]
        s = es[0]
        for e in range(1, N_EXP):
            s = s + es[e]
        r = 1.0 / s
        plsum = jnp.zeros((L,), jnp.float32)
        for e in range(N_EXP):
            p = es[e] * r
            pbuf[e, pl.ds(off, L)] = p
            plsum = plsum + p * ls[e]
        abuf[pl.ds(off, L)] = m - plsum
        sbuf[pl.ds(off, L)] = s
        return carry

    lax.fori_loop(0, GRPS, body, 0)

    for e in range(N_EXP):
        pltpu.sync_copy(pbuf.at[e], probs_hbm.at[e, pl.ds(base, CHUNK)])
    pltpu.sync_copy(abuf, a_hbm.at[pl.ds(base, CHUNK)])
    pltpu.sync_copy(sbuf, s_hbm.at[pl.ds(base, CHUNK)])


def _ent_finish(a_ref, s_ref, out_ref):
    out_ref[0, 0] = jnp.sum(a_ref[...] + jnp.log(s_ref[...]))


def kernel(x, W):
    grid = N_TOKENS // BLK
    logits_t = pl.pallas_call(
        _matmul_blk,
        grid=(grid,),
        in_specs=[
            pl.BlockSpec((BLK, D_MODEL), lambda i: (i, 0)),
            pl.BlockSpec((N_EXP, D_MODEL), lambda i: (0, 0)),
        ],
        out_specs=pl.BlockSpec((N_EXP, BLK), lambda i: (0, i)),
        out_shape=jax.ShapeDtypeStruct((N_EXP, N_TOKENS), jnp.float32),
    )(x, W)

    probs_t, a_part, s_part = _sc_softmax(logits_t)
    logits = logits_t.T
    probs = probs_t.T

    ent_sum = pl.pallas_call(
        _ent_finish,
        out_specs=pl.BlockSpec(memory_space=pltpu.SMEM),
        out_shape=jax.ShapeDtypeStruct((1, 1), jnp.float32),
    )(a_part.reshape(N_TOKENS // 128, 128),
      s_part.reshape(N_TOKENS // 128, 128))
    router_entropy = ent_sum[0, 0] / N_TOKENS
    return (logits, probs, router_entropy)
